# baseline (speedup 1.0000x reference)
import functools

import jax
import jax.numpy as jnp
from jax import lax
from jax.experimental import pallas as pl
from jax.experimental.pallas import tpu as pltpu
from jax.experimental.pallas import tpu_sc as plsc

_NUM_CORES = 2
_NUM_SUBCORES = 16
_NUM_WORKERS = _NUM_CORES * _NUM_SUBCORES

_CHUNK = 128


def _emb_lookup(ids, tables):
    F, B = ids.shape
    _, V, D = tables.shape
    b_per_w = B // _NUM_WORKERS
    n_chunks = b_per_w // _CHUNK

    mesh = plsc.VectorSubcoreMesh(
        core_axis_name="c",
        subcore_axis_name="s",
        num_cores=_NUM_CORES,
        num_subcores=_NUM_SUBCORES,
    )

    @functools.partial(
        pl.kernel,
        mesh=mesh,
        out_type=jax.ShapeDtypeStruct((B, F, D), jnp.float32),
        compiler_params=pltpu.CompilerParams(use_tc_tiling_on_sc=False),
        scratch_types=[
            pltpu.VMEM((_CHUNK,), jnp.int32),
            pltpu.VMEM((_CHUNK, D), jnp.float32),
            pltpu.SemaphoreType.DMA,
        ],
    )
    def body(ids_hbm, tables_hbm, out_hbm, idx_v, rows_v, sem):
        wid = lax.axis_index("s") * _NUM_CORES + lax.axis_index("c")
        base = wid * b_per_w

        def step(t, carry):
            f = t // n_chunks
            c = t % n_chunks
            b0 = base + c * _CHUNK
            pltpu.sync_copy(ids_hbm.at[f, pl.ds(b0, _CHUNK)], idx_v)
            pltpu.async_copy(tables_hbm.at[f].at[idx_v], rows_v, sem).wait()
            pltpu.sync_copy(rows_v, out_hbm.at[pl.ds(b0, _CHUNK), f])
            return carry

        lax.fori_loop(0, F * n_chunks, step, 0)

    return body(ids, tables)


def kernel(ids, tables):
    F, B = ids.shape
    out = _emb_lookup(ids, tables)
    return out.reshape(B, -1)


# speedup vs baseline: 1.0888x; 1.0888x over previous
import jax
import jax.numpy as jnp
from jax import lax
from jax.experimental import pallas as pl
from jax.experimental.pallas import tpu as pltpu
from jax.experimental.pallas import tpu_sc as plsc

_NUM_CORES = 2
_NUM_SUBCORES = 16
_NUM_WORKERS = _NUM_CORES * _NUM_SUBCORES

_CHUNK = 128


def _emb_lookup(ids, tables):
    F, B = ids.shape
    _, V, D = tables.shape
    b_per_w = B // _NUM_WORKERS
    n_chunks = b_per_w // _CHUNK

    mesh = plsc.VectorSubcoreMesh(
        core_axis_name="c",
        subcore_axis_name="s",
        num_cores=_NUM_CORES,
        num_subcores=_NUM_SUBCORES,
    )

    kern = pl.kernel(
        mesh=mesh,
        out_type=jax.ShapeDtypeStruct((B, F, D), jnp.float32),
        compiler_params=pltpu.CompilerParams(use_tc_tiling_on_sc=False),
        scratch_types=[
            pltpu.VMEM((F, b_per_w), jnp.int32),
            pltpu.VMEM((2, b_per_w, D), jnp.float32),
            pltpu.SemaphoreType.DMA,
            pltpu.SemaphoreType.DMA,
            pltpu.SemaphoreType.DMA,
            pltpu.SemaphoreType.DMA,
        ],
    )

    @kern
    def body(ids_hbm, tables_hbm, out_hbm, idx_v, rows_v, g0, g1, w0, w1):
        wid = lax.axis_index("s") * _NUM_CORES + lax.axis_index("c")
        base = wid * b_per_w
        gsems = (g0, g1)
        wsems = (w0, w1)

        pltpu.sync_copy(ids_hbm.at[:, pl.ds(base, b_per_w)], idx_v)

        def fire_gathers(f):
            slot = f % 2
            return [
                pltpu.async_copy(
                    tables_hbm.at[f].at[idx_v.at[f, pl.ds(c * _CHUNK, _CHUNK)]],
                    rows_v.at[slot, pl.ds(c * _CHUNK, _CHUNK)],
                    gsems[slot],
                )
                for c in range(n_chunks)
            ]

        wb = [None, None]
        inflight = fire_gathers(0)
        for f in range(F):
            slot = f % 2
            nxt = slot ^ 1
            if f + 1 < F:
                if wb[nxt] is not None:
                    wb[nxt].wait()
                    wb[nxt] = None
                nxt_gathers = fire_gathers(f + 1)
            else:
                nxt_gathers = None
            for d in inflight:
                d.wait()
            wb[slot] = pltpu.async_copy(
                rows_v.at[slot],
                out_hbm.at[pl.ds(base, b_per_w), f],
                wsems[slot],
            )
            inflight = nxt_gathers
        for d in wb:
            if d is not None:
                d.wait()

    return body(ids, tables)


def kernel(ids, tables):
    F, B = ids.shape
    out = _emb_lookup(ids, tables)
    return out.reshape(B, -1)


# speedup vs baseline: 1.0895x; 1.0007x over previous
import jax
import jax.numpy as jnp
from jax import lax
from jax.experimental import pallas as pl
from jax.experimental.pallas import tpu as pltpu
from jax.experimental.pallas import tpu_sc as plsc

_NUM_CORES = 2
_NUM_SUBCORES = 16
_NUM_WORKERS = _NUM_CORES * _NUM_SUBCORES

_CHUNK = 512


def _emb_lookup(ids, tables):
    F, B = ids.shape
    _, V, D = tables.shape
    b_per_w = B // _NUM_WORKERS
    n_chunks = b_per_w // _CHUNK

    mesh = plsc.VectorSubcoreMesh(
        core_axis_name="c",
        subcore_axis_name="s",
        num_cores=_NUM_CORES,
        num_subcores=_NUM_SUBCORES,
    )

    kern = pl.kernel(
        mesh=mesh,
        out_type=jax.ShapeDtypeStruct((B, F, D), jnp.float32),
        compiler_params=pltpu.CompilerParams(use_tc_tiling_on_sc=False),
        scratch_types=[
            pltpu.VMEM((F, b_per_w), jnp.int32),
            pltpu.VMEM((2, b_per_w, D), jnp.float32),
            pltpu.SemaphoreType.DMA,
            pltpu.SemaphoreType.DMA,
            pltpu.SemaphoreType.DMA,
            pltpu.SemaphoreType.DMA,
        ],
    )

    @kern
    def body(ids_hbm, tables_hbm, out_hbm, idx_v, rows_v, g0, g1, w0, w1):
        wid = lax.axis_index("s") * _NUM_CORES + lax.axis_index("c")
        base = wid * b_per_w
        gsems = (g0, g1)
        wsems = (w0, w1)

        pltpu.sync_copy(ids_hbm.at[:, pl.ds(base, b_per_w)], idx_v)

        def fire_gathers(f):
            slot = f % 2
            return [
                pltpu.async_copy(
                    tables_hbm.at[f].at[idx_v.at[f, pl.ds(c * _CHUNK, _CHUNK)]],
                    rows_v.at[slot, pl.ds(c * _CHUNK, _CHUNK)],
                    gsems[slot],
                )
                for c in range(n_chunks)
            ]

        wb = [None, None]
        inflight = fire_gathers(0)
        for f in range(F):
            slot = f % 2
            nxt = slot ^ 1
            if f + 1 < F:
                if wb[nxt] is not None:
                    wb[nxt].wait()
                    wb[nxt] = None
                nxt_gathers = fire_gathers(f + 1)
            else:
                nxt_gathers = None
            for d in inflight:
                d.wait()
            wb[slot] = pltpu.async_copy(
                rows_v.at[slot],
                out_hbm.at[pl.ds(base, b_per_w), f],
                wsems[slot],
            )
            inflight = nxt_gathers
        for d in wb:
            if d is not None:
                d.wait()

    return body(ids, tables)


def kernel(ids, tables):
    F, B = ids.shape
    out = _emb_lookup(ids, tables)
    return out.reshape(B, -1)
